# baseline (speedup 1.0000x reference)
import functools

import jax
import jax.numpy as jnp
from jax import lax
from jax.experimental import pallas as pl
from jax.experimental.pallas import tpu as pltpu
from jax.experimental.pallas import tpu_sc as plsc

INV_SQRT_2 = 0.5 ** 0.5

_N_TILES = 16
_CHUNK = 512
_NB = _CHUNK // 128
_TRASH = 8


def _make_sc_scatter(n_edges, n_nodes, feat):
    half = n_nodes // 2
    acc_rows = -(-(half + _TRASH) // (_N_TILES * 8)) * (_N_TILES * 8)
    rows_per_tile = acc_rows // _N_TILES
    n_chunks = n_edges // _CHUNK
    iters = -(-n_chunks // _N_TILES)

    mesh = plsc.VectorSubcoreMesh(core_axis_name="c", subcore_axis_name="s")

    @functools.partial(
        pl.kernel,
        out_type=jax.ShapeDtypeStruct((2, 2, acc_rows, feat), jnp.float32),
        mesh=mesh,
        compiler_params=pltpu.CompilerParams(use_tc_tiling_on_sc=False),
        scratch_types=[
            pltpu.VMEM_SHARED((acc_rows, feat), jnp.float32),
            pltpu.VMEM((2, _CHUNK, feat), jnp.float32),
            pltpu.VMEM((2, _NB, 128), jnp.int32),
            pltpu.SemaphoreType.DMA,
            pltpu.SemaphoreType.DMA,
        ],
    )
    def sc_kernel(x_hbm, src_hbm, dst_hbm, zeros_hbm, out_hbm,
                  acc, rows_v, idx_v, sem0, sem1):
        c = lax.axis_index("c")
        t = lax.axis_index("s")
        row0 = t * rows_per_tile
        half_base = c * half
        trash = half + lax.rem(t, _TRASH)
        sems = (sem0, sem1)

        def start_dma(g, p, role_idx_hbm):
            @pl.when(g < n_chunks)
            def _():
                pltpu.async_copy(x_hbm.at[pl.ds(g * _CHUNK, _CHUNK)],
                                 rows_v.at[p], sems[p])
                pltpu.async_copy(role_idx_hbm.at[pl.ds(g * _NB, _NB)],
                                 idx_v.at[p], sems[p])

        def process(g, p):
            @pl.when(g < n_chunks)
            def _():
                pltpu.make_async_copy(x_hbm.at[pl.ds(0, _CHUNK)],
                                      rows_v.at[p], sems[p]).wait()
                pltpu.make_async_copy(src_hbm.at[pl.ds(0, _NB)],
                                      idx_v.at[p], sems[p]).wait()
                for j in range(_NB):
                    for q in range(128 // 16):
                        v = idx_v[p, j, pl.ds(q * 16, 16)]
                        loc = v - half_base
                        ok = (loc >= 0) & (loc < half)
                        idx_v[p, j, pl.ds(q * 16, 16)] = jnp.where(ok, loc,
                                                                   trash)
                for j in range(_NB):
                    pltpu.sync_copy(rows_v.at[p, pl.ds(j * 128, 128)],
                                    acc.at[idx_v.at[p, j]], add=True)

        for role, role_idx_hbm in ((0, src_hbm), (1, dst_hbm)):
            pltpu.sync_copy(zeros_hbm, acc.at[pl.ds(row0, rows_per_tile)])
            plsc.subcore_barrier()

            start_dma(t, 0, role_idx_hbm)

            def chunk_pair(i2, _):
                for p in (0, 1):
                    i = 2 * i2 + p
                    g = t + _N_TILES * i
                    start_dma(g + _N_TILES, 1 - p, role_idx_hbm)
                    process(g, p)

            lax.fori_loop(0, (iters + 1) // 2, chunk_pair, None)
            plsc.subcore_barrier()
            pltpu.sync_copy(acc.at[pl.ds(row0, rows_per_tile)],
                            out_hbm.at[role, c, pl.ds(row0, rows_per_tile)])

    return sc_kernel


def _tc_transpose(x3, n_edges, feat):
    blk_e = 512
    grid = (n_edges // blk_e,)

    def body(x_ref, o_ref):
        a = x_ref[...]
        o_ref[...] = a.reshape(feat, blk_e).T

    return pl.pallas_call(
        body,
        grid=grid,
        in_specs=[pl.BlockSpec((x3.shape[0], x3.shape[1], blk_e),
                               lambda i: (0, 0, i))],
        out_specs=pl.BlockSpec((blk_e, feat), lambda i: (i, 0)),
        out_shape=jax.ShapeDtypeStruct((n_edges, feat), jnp.float32),
    )(x3)


def _tc_transform(acc, k_src, k_dst, coeff, n_nodes):
    blk = 1000
    per_half = (n_nodes // 2) // blk
    grid = (n_nodes // blk,)

    def body(acc_ref, ks_ref, kd_ref, co_ref, out_ref):
        a = acc_ref[...]
        res = jnp.dot(a[0, 0], ks_ref[...],
                      preferred_element_type=jnp.float32)
        res += jnp.dot(a[1, 0], kd_ref[...],
                       preferred_element_type=jnp.float32)
        out_ref[...] = res * co_ref[...]

    return pl.pallas_call(
        body,
        grid=grid,
        in_specs=[
            pl.BlockSpec((2, 1, blk, 48),
                         lambda i: (0, i // per_half, i % per_half, 0)),
            pl.BlockSpec((48, 48), lambda i: (0, 0)),
            pl.BlockSpec((48, 48), lambda i: (0, 0)),
            pl.BlockSpec((blk, 1), lambda i: (i, 0)),
        ],
        out_specs=pl.BlockSpec((blk, 48), lambda i: (i, 0)),
        out_shape=jax.ShapeDtypeStruct((n_nodes, 48), jnp.float32),
    )(acc, k_src, k_dst, coeff)


def kernel(x, src, dst, norm_coeff, n_nodes, W_src, W_dst):
    batch, n_edges, dim_in, dim_k = x.shape
    n_nodes_static = norm_coeff.shape[0]
    feat_total = dim_in * dim_k

    x3 = jnp.transpose(x, (0, 3, 2, 1)).reshape(dim_k, dim_in, n_edges)
    x2d = _tc_transpose(x3, n_edges, feat_total)
    src2d = src.reshape(n_edges // 128, 128)
    dst2d = dst.reshape(n_edges // 128, 128)

    half = n_nodes_static // 2
    rows_per_tile = (-(-(half + _TRASH) // (_N_TILES * 8)) * (_N_TILES * 8)
                     // _N_TILES)

    sc = _make_sc_scatter(n_edges, n_nodes_static, feat_total)
    acc = sc(x2d, src2d, dst2d,
             jnp.zeros((rows_per_tile, feat_total), jnp.float32))

    eye3 = jnp.eye(dim_k, dtype=x.dtype)
    k_src = jnp.einsum('oi,kc->kioc', W_src, eye3).reshape(feat_total,
                                                           feat_total)
    k_dst = jnp.einsum('oi,kc->kioc', W_dst, eye3).reshape(feat_total,
                                                           feat_total)
    scale = jnp.asarray(n_nodes, jnp.float32) / jnp.float32(n_nodes_static)
    coeff = (norm_coeff * (INV_SQRT_2 * scale)).reshape(n_nodes_static, 1)

    out48 = _tc_transform(acc, k_src, k_dst, coeff, n_nodes_static)
    return out48.reshape(batch, n_nodes_static, dim_in, dim_k)


# speedup vs baseline: 1.5544x; 1.5544x over previous
import functools

import jax
import jax.numpy as jnp
from jax import lax
from jax.experimental import pallas as pl
from jax.experimental.pallas import tpu as pltpu
from jax.experimental.pallas import tpu_sc as plsc

INV_SQRT_2 = 0.5 ** 0.5

_N_TILES = 16
_CHUNK = 512
_NB = _CHUNK // 128
_TRASH = 8


def _make_sc_scatter(n_edges, n_nodes, feat):
    half = n_nodes // 2
    acc_rows = -(-(half + _TRASH) // (_N_TILES * 8)) * (_N_TILES * 8)
    rows_per_tile = acc_rows // _N_TILES
    n_chunks = n_edges // _CHUNK
    iters = -(-n_chunks // _N_TILES)

    mesh = plsc.VectorSubcoreMesh(core_axis_name="c", subcore_axis_name="s")

    @functools.partial(
        pl.kernel,
        out_type=jax.ShapeDtypeStruct((2, 2, acc_rows, feat), jnp.float32),
        mesh=mesh,
        compiler_params=pltpu.CompilerParams(use_tc_tiling_on_sc=False),
        scratch_types=[
            pltpu.VMEM_SHARED((acc_rows, feat), jnp.float32),
            pltpu.VMEM((2, _CHUNK, feat), jnp.float32),
            pltpu.VMEM((2, _NB, 128), jnp.int32),
            pltpu.SemaphoreType.DMA,
            pltpu.SemaphoreType.DMA,
        ],
    )
    def sc_kernel(x_hbm, src_hbm, dst_hbm, zeros_hbm, out_hbm,
                  acc, rows_v, idx_v, sem0, sem1):
        c = lax.axis_index("c")
        t = lax.axis_index("s")
        row0 = t * rows_per_tile
        half_base = c * half
        trash = half + lax.rem(t, _TRASH)
        sems = (sem0, sem1)

        def start_dma(g, p, role_idx_hbm):
            @pl.when(g < n_chunks)
            def _():
                pltpu.async_copy(x_hbm.at[pl.ds(g * _CHUNK, _CHUNK)],
                                 rows_v.at[p], sems[p])
                pltpu.async_copy(role_idx_hbm.at[pl.ds(g * _NB, _NB)],
                                 idx_v.at[p], sems[p])

        def process(g, p):
            @pl.when(g < n_chunks)
            def _():
                pltpu.make_async_copy(x_hbm.at[pl.ds(0, _CHUNK)],
                                      rows_v.at[p], sems[p]).wait()
                pltpu.make_async_copy(src_hbm.at[pl.ds(0, _NB)],
                                      idx_v.at[p], sems[p]).wait()
                for j in range(_NB):
                    for q in range(128 // 16):
                        v = idx_v[p, j, pl.ds(q * 16, 16)]
                        loc = v - half_base
                        ok = (loc >= 0) & (loc < half)
                        idx_v[p, j, pl.ds(q * 16, 16)] = jnp.where(ok, loc,
                                                                   trash)
                for j in range(_NB):
                    pltpu.sync_copy(rows_v.at[p, pl.ds(j * 128, 128)],
                                    acc.at[idx_v.at[p, j]], add=True)

        for role, role_idx_hbm in ((0, src_hbm), (1, dst_hbm)):
            pltpu.sync_copy(zeros_hbm, acc.at[pl.ds(row0, rows_per_tile)])
            plsc.subcore_barrier()

            start_dma(t, 0, role_idx_hbm)

            def chunk_pair(i2, _):
                for p in (0, 1):
                    i = 2 * i2 + p
                    g = t + _N_TILES * i
                    start_dma(g + _N_TILES, 1 - p, role_idx_hbm)
                    process(g, p)

            lax.fori_loop(0, (iters + 1) // 2, chunk_pair, None)
            plsc.subcore_barrier()
            pltpu.sync_copy(acc.at[pl.ds(row0, rows_per_tile)],
                            out_hbm.at[role, c, pl.ds(row0, rows_per_tile)])

    return sc_kernel


def _tc_transpose(x3, n_edges, feat):
    blk_e = 2560
    grid = (n_edges // blk_e,)
    eye = jnp.eye(feat, dtype=jnp.float32)

    def body(x_ref, eye_ref, o_ref):
        a2 = x_ref[...].reshape(feat, blk_e)
        o_ref[...] = lax.dot_general(a2, eye_ref[...],
                                     (((0,), (0,)), ((), ())),
                                     preferred_element_type=jnp.float32)

    return pl.pallas_call(
        body,
        grid=grid,
        in_specs=[pl.BlockSpec((x3.shape[0], x3.shape[1], blk_e),
                               lambda i: (0, 0, i)),
                  pl.BlockSpec((feat, feat), lambda i: (0, 0))],
        out_specs=pl.BlockSpec((blk_e, feat), lambda i: (i, 0)),
        out_shape=jax.ShapeDtypeStruct((n_edges, feat), jnp.float32),
    )(x3, eye)


def _tc_transform(acc, k_src, k_dst, coeff, n_nodes):
    blk = 1000
    per_half = (n_nodes // 2) // blk
    grid = (n_nodes // blk,)

    def body(acc_ref, ks_ref, kd_ref, co_ref, out_ref):
        a = acc_ref[...]
        res = jnp.dot(a[0, 0], ks_ref[...],
                      preferred_element_type=jnp.float32)
        res += jnp.dot(a[1, 0], kd_ref[...],
                       preferred_element_type=jnp.float32)
        out_ref[...] = res * co_ref[...]

    return pl.pallas_call(
        body,
        grid=grid,
        in_specs=[
            pl.BlockSpec((2, 1, blk, 48),
                         lambda i: (0, i // per_half, i % per_half, 0)),
            pl.BlockSpec((48, 48), lambda i: (0, 0)),
            pl.BlockSpec((48, 48), lambda i: (0, 0)),
            pl.BlockSpec((blk, 1), lambda i: (i, 0)),
        ],
        out_specs=pl.BlockSpec((blk, 48), lambda i: (i, 0)),
        out_shape=jax.ShapeDtypeStruct((n_nodes, 48), jnp.float32),
    )(acc, k_src, k_dst, coeff)


def kernel(x, src, dst, norm_coeff, n_nodes, W_src, W_dst):
    batch, n_edges, dim_in, dim_k = x.shape
    n_nodes_static = norm_coeff.shape[0]
    feat_total = dim_in * dim_k

    x3 = jnp.transpose(x, (0, 3, 2, 1)).reshape(dim_k, dim_in, n_edges)
    x2d = _tc_transpose(x3, n_edges, feat_total)
    src2d = src.reshape(n_edges // 128, 128)
    dst2d = dst.reshape(n_edges // 128, 128)

    half = n_nodes_static // 2
    rows_per_tile = (-(-(half + _TRASH) // (_N_TILES * 8)) * (_N_TILES * 8)
                     // _N_TILES)

    sc = _make_sc_scatter(n_edges, n_nodes_static, feat_total)
    acc = sc(x2d, src2d, dst2d,
             jnp.zeros((rows_per_tile, feat_total), jnp.float32))

    eye3 = jnp.eye(dim_k, dtype=x.dtype)
    k_src = jnp.einsum('oi,kc->kioc', W_src, eye3).reshape(feat_total,
                                                           feat_total)
    k_dst = jnp.einsum('oi,kc->kioc', W_dst, eye3).reshape(feat_total,
                                                           feat_total)
    scale = jnp.asarray(n_nodes, jnp.float32) / jnp.float32(n_nodes_static)
    coeff = (norm_coeff * (INV_SQRT_2 * scale)).reshape(n_nodes_static, 1)

    out48 = _tc_transform(acc, k_src, k_dst, coeff, n_nodes_static)
    return out48.reshape(batch, n_nodes_static, dim_in, dim_k)


# speedup vs baseline: 1.7272x; 1.1112x over previous
import functools

import jax
import jax.numpy as jnp
from jax import lax
from jax.experimental import pallas as pl
from jax.experimental.pallas import tpu as pltpu
from jax.experimental.pallas import tpu_sc as plsc

INV_SQRT_2 = 0.5 ** 0.5

_N_TILES = 16
_CHUNK = 512
_NB = _CHUNK // 128
_TRASH = 8


def _make_sc_scatter(n_edges, n_nodes, feat):
    half = n_nodes // 2
    acc_rows = -(-(half + _TRASH) // (_N_TILES * 8)) * (_N_TILES * 8)
    rows_per_tile = acc_rows // _N_TILES
    n_chunks = n_edges // _CHUNK
    iters = -(-n_chunks // _N_TILES)

    mesh = plsc.VectorSubcoreMesh(core_axis_name="c", subcore_axis_name="s")

    @functools.partial(
        pl.kernel,
        out_type=jax.ShapeDtypeStruct((2, 2, acc_rows, feat), jnp.float32),
        mesh=mesh,
        compiler_params=pltpu.CompilerParams(use_tc_tiling_on_sc=False),
        scratch_types=[
            pltpu.VMEM_SHARED((acc_rows, feat), jnp.float32),
            pltpu.VMEM((2, _CHUNK, feat), jnp.float32),
            pltpu.VMEM((2, _NB, 128), jnp.int32),
            pltpu.SemaphoreType.DMA,
            pltpu.SemaphoreType.DMA,
        ],
    )
    def sc_kernel(x_hbm, src_hbm, dst_hbm, zeros_hbm, out_hbm,
                  acc, rows_v, idx_v, sem0, sem1):
        c = lax.axis_index("c")
        t = lax.axis_index("s")
        row0 = t * rows_per_tile
        half_base = c * half
        trash = half + lax.rem(t, _TRASH)
        sems = (sem0, sem1)

        def start_dma(g, p, role_idx_hbm):
            @pl.when(g < n_chunks)
            def _():
                pltpu.async_copy(x_hbm.at[pl.ds(g * _CHUNK, _CHUNK)],
                                 rows_v.at[p], sems[p])
                pltpu.async_copy(role_idx_hbm.at[pl.ds(g * _NB, _NB)],
                                 idx_v.at[p], sems[p])

        def process(g, p):
            @pl.when(g < n_chunks)
            def _():
                pltpu.make_async_copy(x_hbm.at[pl.ds(0, _CHUNK)],
                                      rows_v.at[p], sems[p]).wait()
                pltpu.make_async_copy(src_hbm.at[pl.ds(0, _NB)],
                                      idx_v.at[p], sems[p]).wait()
                for j in range(_NB):
                    for q in range(128 // 16):
                        v = idx_v[p, j, pl.ds(q * 16, 16)]
                        loc = v - half_base
                        ok = (loc >= 0) & (loc < half)
                        idx_v[p, j, pl.ds(q * 16, 16)] = jnp.where(ok, loc,
                                                                   trash)
                for j in range(_NB):
                    pltpu.sync_copy(rows_v.at[p, pl.ds(j * 128, 128)],
                                    acc.at[idx_v.at[p, j]], add=True)

        for role, role_idx_hbm in ((0, src_hbm), (1, dst_hbm)):
            pltpu.sync_copy(zeros_hbm, acc.at[pl.ds(row0, rows_per_tile)])
            plsc.subcore_barrier()

            start_dma(t, 0, role_idx_hbm)

            def chunk_pair(i2, _):
                for p in (0, 1):
                    i = 2 * i2 + p
                    g = t + _N_TILES * i
                    start_dma(g + _N_TILES, 1 - p, role_idx_hbm)
                    process(g, p)

            lax.fori_loop(0, (iters + 1) // 2, chunk_pair, None)
            plsc.subcore_barrier()
            pltpu.sync_copy(acc.at[pl.ds(row0, rows_per_tile)],
                            out_hbm.at[role, c, pl.ds(row0, rows_per_tile)])

    return sc_kernel


def _tc_transpose(x48, n_edges, feat):
    blk_e = 6400
    grid = (n_edges // blk_e,)
    eye = jnp.eye(feat, dtype=jnp.float32)

    def body(x_ref, eye_ref, o_ref):
        o_ref[...] = lax.dot_general(x_ref[...], eye_ref[...],
                                     (((0,), (0,)), ((), ())),
                                     preferred_element_type=jnp.float32)

    return pl.pallas_call(
        body,
        grid=grid,
        in_specs=[pl.BlockSpec((feat, blk_e), lambda i: (0, i)),
                  pl.BlockSpec((feat, feat), lambda i: (0, 0))],
        out_specs=pl.BlockSpec((blk_e, feat), lambda i: (i, 0)),
        out_shape=jax.ShapeDtypeStruct((n_edges, feat), jnp.float32),
    )(x48, eye)


def _tc_transform(acc, k_src, k_dst, coeff, n_nodes):
    blk = 1000
    per_half = (n_nodes // 2) // blk
    grid = (n_nodes // blk,)

    def body(acc_ref, ks_ref, kd_ref, co_ref, out_ref):
        a = acc_ref[...]
        res = jnp.dot(a[0, 0], ks_ref[...],
                      preferred_element_type=jnp.float32)
        res += jnp.dot(a[1, 0], kd_ref[...],
                       preferred_element_type=jnp.float32)
        out_ref[...] = res * co_ref[...]

    return pl.pallas_call(
        body,
        grid=grid,
        in_specs=[
            pl.BlockSpec((2, 1, blk, 48),
                         lambda i: (0, i // per_half, i % per_half, 0)),
            pl.BlockSpec((48, 48), lambda i: (0, 0)),
            pl.BlockSpec((48, 48), lambda i: (0, 0)),
            pl.BlockSpec((blk, 1), lambda i: (i, 0)),
        ],
        out_specs=pl.BlockSpec((blk, 48), lambda i: (i, 0)),
        out_shape=jax.ShapeDtypeStruct((n_nodes, 48), jnp.float32),
    )(acc, k_src, k_dst, coeff)


def kernel(x, src, dst, norm_coeff, n_nodes, W_src, W_dst):
    batch, n_edges, dim_in, dim_k = x.shape
    n_nodes_static = norm_coeff.shape[0]
    feat_total = dim_in * dim_k

    x48 = jnp.transpose(x, (0, 3, 2, 1)).reshape(feat_total, n_edges)
    x2d = _tc_transpose(x48, n_edges, feat_total)
    src2d = src.reshape(n_edges // 128, 128)
    dst2d = dst.reshape(n_edges // 128, 128)

    half = n_nodes_static // 2
    rows_per_tile = (-(-(half + _TRASH) // (_N_TILES * 8)) * (_N_TILES * 8)
                     // _N_TILES)

    sc = _make_sc_scatter(n_edges, n_nodes_static, feat_total)
    acc = sc(x2d, src2d, dst2d,
             jnp.zeros((rows_per_tile, feat_total), jnp.float32))

    eye3 = jnp.eye(dim_k, dtype=x.dtype)
    k_src = jnp.einsum('oi,kc->kioc', W_src, eye3).reshape(feat_total,
                                                           feat_total)
    k_dst = jnp.einsum('oi,kc->kioc', W_dst, eye3).reshape(feat_total,
                                                           feat_total)
    scale = jnp.asarray(n_nodes, jnp.float32) / jnp.float32(n_nodes_static)
    coeff = (norm_coeff * (INV_SQRT_2 * scale)).reshape(n_nodes_static, 1)

    out48 = _tc_transform(acc, k_src, k_dst, coeff, n_nodes_static)
    return out48.reshape(batch, n_nodes_static, dim_in, dim_k)


# speedup vs baseline: 2.6836x; 1.5537x over previous
import functools

import jax
import jax.numpy as jnp
from jax import lax
from jax.experimental import pallas as pl
from jax.experimental.pallas import tpu as pltpu
from jax.experimental.pallas import tpu_sc as plsc

INV_SQRT_2 = 0.5 ** 0.5

_N_TILES = 16
_CHUNK = 512
_NB = _CHUNK // 128
_TRASH = 8


def _make_sc_scatter(n_edges, n_nodes, feat):
    half = n_nodes // 2
    acc_rows = -(-(half + _TRASH) // (_N_TILES * 8)) * (_N_TILES * 8)
    rows_per_tile = acc_rows // _N_TILES
    n_chunks = n_edges // _CHUNK
    iters = -(-n_chunks // _N_TILES)

    mesh = plsc.VectorSubcoreMesh(core_axis_name="c", subcore_axis_name="s")

    @functools.partial(
        pl.kernel,
        out_type=jax.ShapeDtypeStruct((2, 2, acc_rows, feat), jnp.float32),
        mesh=mesh,
        compiler_params=pltpu.CompilerParams(use_tc_tiling_on_sc=False),
        scratch_types=[
            pltpu.VMEM_SHARED((acc_rows, feat), jnp.float32),
            pltpu.VMEM((2, _CHUNK, feat), jnp.float32),
            pltpu.VMEM((2, _NB, 128), jnp.int32),
            pltpu.SemaphoreType.DMA,
            pltpu.SemaphoreType.DMA,
        ],
    )
    def sc_kernel(x_hbm, src_hbm, dst_hbm, zeros_hbm, out_hbm,
                  acc, rows_v, idx_v, sem0, sem1):
        c = lax.axis_index("c")
        t = lax.axis_index("s")
        row0 = t * rows_per_tile
        half_base = c * half
        trash = half + lax.rem(t, _TRASH)
        sems = (sem0, sem1)

        def start_dma(g, p, role_idx_hbm):
            @pl.when(g < n_chunks)
            def _():
                pltpu.async_copy(x_hbm.at[pl.ds(g * _CHUNK, _CHUNK),
                                          pl.ds(0, feat)],
                                 rows_v.at[p], sems[p])
                pltpu.async_copy(role_idx_hbm.at[pl.ds(g * _NB, _NB)],
                                 idx_v.at[p], sems[p])

        def process(g, p):
            @pl.when(g < n_chunks)
            def _():
                pltpu.make_async_copy(x_hbm.at[pl.ds(0, _CHUNK),
                                               pl.ds(0, feat)],
                                      rows_v.at[p], sems[p]).wait()
                pltpu.make_async_copy(src_hbm.at[pl.ds(0, _NB)],
                                      idx_v.at[p], sems[p]).wait()
                for j in range(_NB):
                    for q in range(128 // 16):
                        v = idx_v[p, j, pl.ds(q * 16, 16)]
                        loc = v - half_base
                        ok = (loc >= 0) & (loc < half)
                        idx_v[p, j, pl.ds(q * 16, 16)] = jnp.where(ok, loc,
                                                                   trash)
                for j in range(_NB):
                    pltpu.sync_copy(rows_v.at[p, pl.ds(j * 128, 128)],
                                    acc.at[idx_v.at[p, j]], add=True)

        for role, role_idx_hbm in ((0, src_hbm), (1, dst_hbm)):
            pltpu.sync_copy(zeros_hbm, acc.at[pl.ds(row0, rows_per_tile)])
            plsc.subcore_barrier()

            start_dma(t, 0, role_idx_hbm)

            def chunk_pair(i2, _):
                for p in (0, 1):
                    i = 2 * i2 + p
                    g = t + _N_TILES * i
                    start_dma(g + _N_TILES, 1 - p, role_idx_hbm)
                    process(g, p)

            lax.fori_loop(0, (iters + 1) // 2, chunk_pair, None)
            plsc.subcore_barrier()
            pltpu.sync_copy(acc.at[pl.ds(row0, rows_per_tile)],
                            out_hbm.at[role, c, pl.ds(row0, rows_per_tile)])

    return sc_kernel


def _tc_transpose(x48, n_edges, feat):
    blk_e = 6400
    grid = (n_edges // blk_e,)
    eye_pad = jnp.eye(feat, 128, dtype=jnp.float32)

    def body(x_ref, eye_ref, o_ref):
        o_ref[...] = lax.dot_general(x_ref[...], eye_ref[...],
                                     (((0,), (0,)), ((), ())),
                                     preferred_element_type=jnp.float32)

    return pl.pallas_call(
        body,
        grid=grid,
        in_specs=[pl.BlockSpec((feat, blk_e), lambda i: (0, i)),
                  pl.BlockSpec((feat, 128), lambda i: (0, 0))],
        out_specs=pl.BlockSpec((blk_e, 128), lambda i: (i, 0)),
        out_shape=jax.ShapeDtypeStruct((n_edges, 128), jnp.float32),
    )(x48, eye_pad)


def _tc_transform(acc, k_src, k_dst, coeff, n_nodes):
    blk = 1000
    per_half = (n_nodes // 2) // blk
    grid = (n_nodes // blk,)

    def body(acc_ref, ks_ref, kd_ref, co_ref, out_ref):
        a = acc_ref[...]
        res = jnp.dot(a[0, 0], ks_ref[...],
                      preferred_element_type=jnp.float32)
        res += jnp.dot(a[1, 0], kd_ref[...],
                       preferred_element_type=jnp.float32)
        out_ref[...] = res * co_ref[...]

    return pl.pallas_call(
        body,
        grid=grid,
        in_specs=[
            pl.BlockSpec((2, 1, blk, 48),
                         lambda i: (0, i // per_half, i % per_half, 0)),
            pl.BlockSpec((48, 48), lambda i: (0, 0)),
            pl.BlockSpec((48, 48), lambda i: (0, 0)),
            pl.BlockSpec((blk, 1), lambda i: (i, 0)),
        ],
        out_specs=pl.BlockSpec((blk, 48), lambda i: (i, 0)),
        out_shape=jax.ShapeDtypeStruct((n_nodes, 48), jnp.float32),
    )(acc, k_src, k_dst, coeff)


def kernel(x, src, dst, norm_coeff, n_nodes, W_src, W_dst):
    batch, n_edges, dim_in, dim_k = x.shape
    n_nodes_static = norm_coeff.shape[0]
    feat_total = dim_in * dim_k

    x48 = jnp.transpose(x, (0, 3, 2, 1)).reshape(feat_total, n_edges)
    x2d = _tc_transpose(x48, n_edges, feat_total)
    src2d = src.reshape(n_edges // 128, 128)
    dst2d = dst.reshape(n_edges // 128, 128)

    half = n_nodes_static // 2
    rows_per_tile = (-(-(half + _TRASH) // (_N_TILES * 8)) * (_N_TILES * 8)
                     // _N_TILES)

    sc = _make_sc_scatter(n_edges, n_nodes_static, feat_total)
    acc = sc(x2d, src2d, dst2d,
             jnp.zeros((rows_per_tile, feat_total), jnp.float32))

    eye3 = jnp.eye(dim_k, dtype=x.dtype)
    k_src = jnp.einsum('oi,kc->kioc', W_src, eye3).reshape(feat_total,
                                                           feat_total)
    k_dst = jnp.einsum('oi,kc->kioc', W_dst, eye3).reshape(feat_total,
                                                           feat_total)
    scale = jnp.asarray(n_nodes, jnp.float32) / jnp.float32(n_nodes_static)
    coeff = (norm_coeff * (INV_SQRT_2 * scale)).reshape(n_nodes_static, 1)

    out48 = _tc_transform(acc, k_src, k_dst, coeff, n_nodes_static)
    return out48.reshape(batch, n_nodes_static, dim_in, dim_k)


# speedup vs baseline: 2.6892x; 1.0021x over previous
import functools

import jax
import jax.numpy as jnp
from jax import lax
from jax.experimental import pallas as pl
from jax.experimental.pallas import tpu as pltpu
from jax.experimental.pallas import tpu_sc as plsc

INV_SQRT_2 = 0.5 ** 0.5

_N_TILES = 16
_CHUNK = 512
_NB = _CHUNK // 128
_TRASH = 8


def _make_sc_scatter(n_edges, n_nodes, feat):
    half = n_nodes // 2
    acc_rows = -(-(half + _TRASH) // (_N_TILES * 8)) * (_N_TILES * 8)
    rows_per_tile = acc_rows // _N_TILES
    n_chunks = n_edges // _CHUNK
    iters = -(-n_chunks // _N_TILES)

    mesh = plsc.VectorSubcoreMesh(core_axis_name="c", subcore_axis_name="s")

    @functools.partial(
        pl.kernel,
        out_type=jax.ShapeDtypeStruct((2, 2, acc_rows, feat), jnp.float32),
        mesh=mesh,
        compiler_params=pltpu.CompilerParams(use_tc_tiling_on_sc=False),
        scratch_types=[
            pltpu.VMEM_SHARED((acc_rows, feat), jnp.float32),
            pltpu.VMEM((2, _CHUNK, feat), jnp.float32),
            pltpu.VMEM((2, _NB, 128), jnp.int32),
            pltpu.SemaphoreType.DMA,
            pltpu.SemaphoreType.DMA,
            pltpu.SemaphoreType.DMA,
            pltpu.SemaphoreType.DMA,
        ],
    )
    def sc_kernel(x_hbm, src_hbm, dst_hbm, zeros_hbm, out_hbm,
                  acc, rows_v, idx_v, sem0, sem1, ssem0, ssem1):
        c = lax.axis_index("c")
        t = lax.axis_index("s")
        row0 = t * rows_per_tile
        half_base = c * half
        trash = half + lax.rem(t, _TRASH)
        sems = (sem0, sem1)
        ssems = (ssem0, ssem1)

        def start_dma(g, p, role_idx_hbm):
            @pl.when(g < n_chunks)
            def _():
                pltpu.async_copy(x_hbm.at[pl.ds(g * _CHUNK, _CHUNK),
                                          pl.ds(0, feat)],
                                 rows_v.at[p], sems[p])
                pltpu.async_copy(role_idx_hbm.at[pl.ds(g * _NB, _NB)],
                                 idx_v.at[p], sems[p])

        def drain_scatters(g, p):
            @pl.when((g >= 0) & (g < n_chunks))
            def _():
                for j in range(_NB):
                    pltpu.make_async_copy(rows_v.at[p, pl.ds(j * 128, 128)],
                                          acc.at[idx_v.at[p, j]],
                                          ssems[p]).wait()

        def process(g, p):
            @pl.when(g < n_chunks)
            def _():
                pltpu.make_async_copy(x_hbm.at[pl.ds(0, _CHUNK),
                                               pl.ds(0, feat)],
                                      rows_v.at[p], sems[p]).wait()
                pltpu.make_async_copy(src_hbm.at[pl.ds(0, _NB)],
                                      idx_v.at[p], sems[p]).wait()
                for j in range(_NB):
                    for q in range(128 // 16):
                        v = idx_v[p, j, pl.ds(q * 16, 16)]
                        loc = v - half_base
                        ok = (loc >= 0) & (loc < half)
                        idx_v[p, j, pl.ds(q * 16, 16)] = jnp.where(ok, loc,
                                                                   trash)
                for j in range(_NB):
                    pltpu.async_copy(rows_v.at[p, pl.ds(j * 128, 128)],
                                     acc.at[idx_v.at[p, j]], ssems[p],
                                     add=True)

        for role, role_idx_hbm in ((0, src_hbm), (1, dst_hbm)):
            pltpu.sync_copy(zeros_hbm, acc.at[pl.ds(row0, rows_per_tile)])
            plsc.subcore_barrier()

            start_dma(t, 0, role_idx_hbm)

            def chunk_pair(i2, _):
                for p in (0, 1):
                    i = 2 * i2 + p
                    g = t + _N_TILES * i
                    drain_scatters(g - _N_TILES, 1 - p)
                    start_dma(g + _N_TILES, 1 - p, role_idx_hbm)
                    process(g, p)

            lax.fori_loop(0, (iters + 1) // 2, chunk_pair, None)
            g_last = t + _N_TILES * (iters - 1)
            drain_scatters(g_last, (iters - 1) % 2)
            plsc.subcore_barrier()
            pltpu.sync_copy(acc.at[pl.ds(row0, rows_per_tile)],
                            out_hbm.at[role, c, pl.ds(row0, rows_per_tile)])

    return sc_kernel


def _tc_transpose(x48, n_edges, feat):
    blk_e = 6400
    grid = (n_edges // blk_e,)
    eye_pad = jnp.eye(feat, 128, dtype=jnp.float32)

    def body(x_ref, eye_ref, o_ref):
        o_ref[...] = lax.dot_general(x_ref[...], eye_ref[...],
                                     (((0,), (0,)), ((), ())),
                                     preferred_element_type=jnp.float32)

    return pl.pallas_call(
        body,
        grid=grid,
        in_specs=[pl.BlockSpec((feat, blk_e), lambda i: (0, i)),
                  pl.BlockSpec((feat, 128), lambda i: (0, 0))],
        out_specs=pl.BlockSpec((blk_e, 128), lambda i: (i, 0)),
        out_shape=jax.ShapeDtypeStruct((n_edges, 128), jnp.float32),
    )(x48, eye_pad)


def _tc_transform(acc, k_src, k_dst, coeff, n_nodes):
    blk = 1000
    per_half = (n_nodes // 2) // blk
    grid = (n_nodes // blk,)

    def body(acc_ref, ks_ref, kd_ref, co_ref, out_ref):
        a = acc_ref[...]
        res = jnp.dot(a[0, 0], ks_ref[...],
                      preferred_element_type=jnp.float32)
        res += jnp.dot(a[1, 0], kd_ref[...],
                       preferred_element_type=jnp.float32)
        out_ref[...] = res * co_ref[...]

    return pl.pallas_call(
        body,
        grid=grid,
        in_specs=[
            pl.BlockSpec((2, 1, blk, 48),
                         lambda i: (0, i // per_half, i % per_half, 0)),
            pl.BlockSpec((48, 48), lambda i: (0, 0)),
            pl.BlockSpec((48, 48), lambda i: (0, 0)),
            pl.BlockSpec((blk, 1), lambda i: (i, 0)),
        ],
        out_specs=pl.BlockSpec((blk, 48), lambda i: (i, 0)),
        out_shape=jax.ShapeDtypeStruct((n_nodes, 48), jnp.float32),
    )(acc, k_src, k_dst, coeff)


def kernel(x, src, dst, norm_coeff, n_nodes, W_src, W_dst):
    batch, n_edges, dim_in, dim_k = x.shape
    n_nodes_static = norm_coeff.shape[0]
    feat_total = dim_in * dim_k

    x48 = jnp.transpose(x, (0, 3, 2, 1)).reshape(feat_total, n_edges)
    x2d = _tc_transpose(x48, n_edges, feat_total)
    src2d = src.reshape(n_edges // 128, 128)
    dst2d = dst.reshape(n_edges // 128, 128)

    half = n_nodes_static // 2
    rows_per_tile = (-(-(half + _TRASH) // (_N_TILES * 8)) * (_N_TILES * 8)
                     // _N_TILES)

    sc = _make_sc_scatter(n_edges, n_nodes_static, feat_total)
    acc = sc(x2d, src2d, dst2d,
             jnp.zeros((rows_per_tile, feat_total), jnp.float32))

    eye3 = jnp.eye(dim_k, dtype=x.dtype)
    k_src = jnp.einsum('oi,kc->kioc', W_src, eye3).reshape(feat_total,
                                                           feat_total)
    k_dst = jnp.einsum('oi,kc->kioc', W_dst, eye3).reshape(feat_total,
                                                           feat_total)
    scale = jnp.asarray(n_nodes, jnp.float32) / jnp.float32(n_nodes_static)
    coeff = (norm_coeff * (INV_SQRT_2 * scale)).reshape(n_nodes_static, 1)

    out48 = _tc_transform(acc, k_src, k_dst, coeff, n_nodes_static)
    return out48.reshape(batch, n_nodes_static, dim_in, dim_k)
